# baseline (speedup 1.0000x reference)
import functools
import math

import jax
import jax.numpy as jnp
from jax import lax
from jax.experimental import pallas as pl
from jax.experimental.pallas import tpu as pltpu
from jax.experimental.pallas import tpu_sc as plsc

D_MODEL = 64
SCALE = math.sqrt(D_MODEL)

_info = plsc.get_sparse_core_info()
_NC, _NS, _L = _info.num_cores, _info.num_subcores, _info.num_lanes
_NW = _NC * _NS


@functools.partial(jax.jit, static_argnames=("b_total", "d"))
def _emb_lookup(x3, table, b_total, d):
    rpg = 8
    n_groups = table.shape[0] // rpg
    b_per_w = b_total // _NW
    nb = _L
    ch = 64
    n_ch = b_per_w // ch
    mesh = plsc.VectorSubcoreMesh(core_axis_name="c", subcore_axis_name="s")

    @functools.partial(
        pl.kernel,
        mesh=mesh,
        out_type=jax.ShapeDtypeStruct((b_total, d), jnp.float32),
        scratch_types=[
            pltpu.VMEM((n_ch, ch), jnp.int32),
            pltpu.VMEM((nb, rpg, d), jnp.float32),
            pltpu.VMEM((ch, d), jnp.float32),
            pltpu.SemaphoreType.DMA,
        ],
        compiler_params=pltpu.CompilerParams(use_tc_tiling_on_sc=True),
    )
    def k(idx_hbm, tab_hbm, out_hbm, idx_v, buf_v, stage_v, sem):
        wid = lax.axis_index("s") * _NC + lax.axis_index("c")
        base = wid * b_per_w
        tab3 = tab_hbm.reshape(n_groups, rpg, d)
        pltpu.sync_copy(idx_hbm.at[wid], idx_v)

        def chunk_body(c, carry):
            def batch_body(b, carry2):
                idxvec = idx_v[c, pl.ds(b * nb, nb)]
                gvec = idxvec >> 3
                offvec = idxvec & 7
                for s in range(nb):
                    pltpu.async_copy(tab3.at[gvec[s]], buf_v.at[s], sem)
                for s in range(nb):
                    pltpu.make_async_copy(tab3.at[0], buf_v.at[s], sem).wait()
                for s in range(nb):
                    r = b * nb + s
                    for kk in range(d // _L):
                        sl = pl.ds(kk * _L, _L)
                        stage_v[r, sl] = buf_v[s, offvec[s], sl] * SCALE
                return carry2

            lax.fori_loop(0, ch // nb, batch_body, 0)
            pltpu.sync_copy(stage_v, out_hbm.at[pl.ds(base + c * ch, ch)])
            return carry

        lax.fori_loop(0, n_ch, chunk_body, 0)

    return k(x3, table)


def kernel(x, weight):
    b0, b1 = x.shape
    b_total = b0 * b1
    b_per_w = b_total // _NW
    x3 = x.astype(jnp.int32).reshape(_NW, b_per_w // 64, 64)
    out = _emb_lookup(x3, weight, b_total, D_MODEL)
    return out.reshape(b0, b1, D_MODEL)
